# baseline (speedup 1.0000x reference)
import jax
import jax.numpy as jnp
from jax import lax
from jax.experimental import pallas as pl
from jax.experimental.pallas import tpu as pltpu
from jax.experimental.pallas import tpu_sc as plsc

_F = 128
_Q = 65536
_B = 4096

_NC = 2
_NS = 16
_NW = _NC * _NS
_L = 16

_CPW = _B // _NW
_RPW = _F // _NW
_TP = 136


def _sc_body(feat_hbm, q_hbm, out_hbm, fbuf, tbuf, csem, fsem, osem):
    cid = lax.axis_index("c")
    sid = lax.axis_index("s")
    wid = sid * _NC + cid

    r0 = wid * _RPW
    c0 = wid * _CPW

    cps = [
        pltpu.async_copy(
            q_hbm.at[pl.ds((r0 + i) * _Q + _B, _Q - _B)],
            out_hbm.at[pl.ds((r0 + i) * _Q + _B, _Q - _B)],
            csem,
        )
        for i in range(_RPW)
    ]

    pltpu.async_copy(feat_hbm.at[pl.ds(c0 * _F, _CPW * _F)], fbuf, fsem).wait()

    iot = jnp.arange(_L, dtype=jnp.int32)

    def _tbody(m, carry):
        j0 = (m // 8) * _L
        f0 = (m % 8) * _L
        for d in range(_L):
            row = j0 + ((iot + d) & (_L - 1))
            col = f0 + iot
            v = plsc.load_gather(fbuf, [row * _F + col])
            plsc.store_scatter(tbuf, [col * _TP + row], v)
        return carry

    lax.fori_loop(0, (_CPW // _L) * (_F // _L), _tbody, 0)

    ods = [
        pltpu.async_copy(
            tbuf.at[pl.ds(f * _TP, _CPW)],
            out_hbm.at[pl.ds(f * _Q + c0, _CPW)],
            osem,
        )
        for f in range(_F)
    ]
    for d in ods:
        d.wait()

    for cp in cps:
        cp.wait()


def kernel(features, queue):
    fn = pl.kernel(
        _sc_body,
        out_type=jax.ShapeDtypeStruct((_F * _Q,), jnp.float32),
        mesh=plsc.VectorSubcoreMesh(core_axis_name="c", subcore_axis_name="s"),
        compiler_params=pltpu.CompilerParams(needs_layout_passes=False),
        scratch_types=[
            pltpu.VMEM((_CPW * _F,), jnp.float32),
            pltpu.VMEM((_F * _TP,), jnp.float32),
            pltpu.SemaphoreType.DMA,
            pltpu.SemaphoreType.DMA,
            pltpu.SemaphoreType.DMA,
        ],
    )
    return fn(features.reshape(-1), queue.reshape(-1)).reshape(_F, _Q)


# speedup vs baseline: 18.4836x; 18.4836x over previous
import jax
import jax.numpy as jnp
from jax import lax
from jax.experimental import pallas as pl
from jax.experimental.pallas import tpu as pltpu
from jax.experimental.pallas import tpu_sc as plsc

_F = 128
_Q = 65536
_B = 4096

_NC = 2
_NS = 16
_NW = _NC * _NS
_L = 16

_CPW = _B // _NW
_NB = 16
_BH = _F // _NB
_PAD = 137

_HALF = (_Q - _B) // 2
_CCH = 2560
_NCH = _HALF // _CCH
_NBUF = 3


def _sc_body(feat_hbm, q_hbm, out_hbm, fbuf, tbuf, cb0, cb1, cb2,
             fsem, osem, rsem, wsem):
    cid = lax.axis_index("c")
    sid = lax.axis_index("s")
    wid = sid * _NC + cid

    c0 = wid * _CPW
    fl = pltpu.async_copy(
        feat_hbm.at[pl.ds(c0, _CPW), :], fbuf.at[:, pl.ds(0, _F)], fsem
    )

    band = (wid % _NB) * _BH
    col0 = _B + (wid // _NB) * _HALF
    cbufs = [cb0, cb1, cb2]

    def _src(k):
        return q_hbm.at[pl.ds(band, _BH), pl.ds(col0 + k * _CCH, _CCH)]

    def _dst(k):
        return out_hbm.at[pl.ds(band, _BH), pl.ds(col0 + k * _CCH, _CCH)]

    rd = {}
    wd = {}
    for i in range(_NBUF):
        rd[i] = pltpu.async_copy(_src(i), cbufs[i], rsem)

    fl.wait()
    iot = jnp.arange(_L, dtype=jnp.int32)

    def _tbody(t, carry):
        b = t // _BH
        r = t % _BH
        base_v = iot * 0 + (b * _BH + r)
        for lb in range(_CPW // _L):
            v = plsc.load_gather(fbuf, [lb * _L + iot, base_v])
            tbuf[b, r, pl.ds(lb * _L, _L)] = v
        return carry

    lax.fori_loop(0, _NB * _BH, _tbody, 0)

    ods = [
        pltpu.async_copy(
            tbuf.at[b],
            out_hbm.at[pl.ds(b * _BH, _BH), pl.ds(c0, _CPW)],
            osem,
        )
        for b in range(_NB)
    ]

    for k in range(_NCH):
        s = k % _NBUF
        if k >= _NBUF:
            wd[s].wait()
            rd[s] = pltpu.async_copy(_src(k), cbufs[s], rsem)
        rd[s].wait()
        wd[s] = pltpu.async_copy(cbufs[s], _dst(k), wsem)

    for k in range(_NCH - _NBUF, _NCH):
        wd[k % _NBUF].wait()
    for d in ods:
        d.wait()


def kernel(features, queue):
    fn = pl.kernel(
        _sc_body,
        out_type=jax.ShapeDtypeStruct((_F, _Q), jnp.float32),
        mesh=plsc.VectorSubcoreMesh(core_axis_name="c", subcore_axis_name="s"),
        compiler_params=pltpu.CompilerParams(needs_layout_passes=False),
        scratch_types=[
            pltpu.VMEM((_CPW, _PAD), jnp.float32),
            pltpu.VMEM((_NB, _BH, _CPW), jnp.float32),
            pltpu.VMEM((_BH, _CCH), jnp.float32),
            pltpu.VMEM((_BH, _CCH), jnp.float32),
            pltpu.VMEM((_BH, _CCH), jnp.float32),
            pltpu.SemaphoreType.DMA,
            pltpu.SemaphoreType.DMA,
            pltpu.SemaphoreType.DMA,
            pltpu.SemaphoreType.DMA,
        ],
    )
    return fn(features, queue)


# speedup vs baseline: 18.8695x; 1.0209x over previous
import jax
import jax.numpy as jnp
from jax import lax
from jax.experimental import pallas as pl
from jax.experimental.pallas import tpu as pltpu
from jax.experimental.pallas import tpu_sc as plsc

_F = 128
_Q = 65536
_B = 4096

_NC = 2
_NS = 16
_NW = _NC * _NS
_L = 16

_CPW = _B // _NW
_NB = 16
_BH = _F // _NB
_PAD = 137

_HALF = (_Q - _B) // 2
_CCH = 1536
_NCH = _HALF // _CCH
_NBUF = 6
_AHEAD = 2


def _sc_body(feat_hbm, q_hbm, out_hbm, fbuf, tbuf, cbufs,
             fsem, osem, rsem, wsem):
    cid = lax.axis_index("c")
    sid = lax.axis_index("s")
    wid = sid * _NC + cid

    c0 = wid * _CPW
    fl = pltpu.async_copy(
        feat_hbm.at[pl.ds(c0, _CPW), :], fbuf.at[:, pl.ds(0, _F)], fsem
    )

    band = (wid % _NB) * _BH
    col0 = _B + (wid // _NB) * _HALF

    def _src(k):
        return q_hbm.at[pl.ds(band, _BH), pl.ds(col0 + k * _CCH, _CCH)]

    def _dst(k):
        return out_hbm.at[pl.ds(band, _BH), pl.ds(col0 + k * _CCH, _CCH)]

    rd = {}
    wd = {}
    for i in range(_AHEAD + 1):
        rd[i] = pltpu.async_copy(_src(i), cbufs[i], rsem)

    fl.wait()
    iot = jnp.arange(_L, dtype=jnp.int32)

    def _tbody(t, carry):
        b = t // _BH
        r = t % _BH
        base_v = iot * 0 + (b * _BH + r)
        for lb in range(_CPW // _L):
            v = plsc.load_gather(fbuf, [lb * _L + iot, base_v])
            tbuf[b, r, pl.ds(lb * _L, _L)] = v
        return carry

    lax.fori_loop(0, _NB * _BH, _tbody, 0)

    ods = [
        pltpu.async_copy(
            tbuf.at[b],
            out_hbm.at[pl.ds(b * _BH, _BH), pl.ds(c0, _CPW)],
            osem,
        )
        for b in range(_NB)
    ]

    for k in range(_NCH):
        s = k % _NBUF
        kn = k + _AHEAD + 1
        if kn < _NCH:
            sn = kn % _NBUF
            if kn >= _NBUF:
                wd[sn].wait()
            rd[sn] = pltpu.async_copy(_src(kn), cbufs[sn], rsem)
        rd[s].wait()
        wd[s] = pltpu.async_copy(cbufs[s], _dst(k), wsem)

    for k in range(max(0, _NCH - _NBUF), _NCH):
        wd[k % _NBUF].wait()
    for d in ods:
        d.wait()


def kernel(features, queue):
    fn = pl.kernel(
        _sc_body,
        out_type=jax.ShapeDtypeStruct((_F, _Q), jnp.float32),
        mesh=plsc.VectorSubcoreMesh(core_axis_name="c", subcore_axis_name="s"),
        compiler_params=pltpu.CompilerParams(needs_layout_passes=False),
        scratch_types=[
            pltpu.VMEM((_CPW, _PAD), jnp.float32),
            pltpu.VMEM((_NB, _BH, _CPW), jnp.float32),
            [pltpu.VMEM((_BH, _CCH), jnp.float32) for _ in range(_NBUF)],
            pltpu.SemaphoreType.DMA,
            pltpu.SemaphoreType.DMA,
            pltpu.SemaphoreType.DMA,
            pltpu.SemaphoreType.DMA,
        ],
    )
    return fn(features, queue)
